# baseline (speedup 1.0000x reference)
import functools

import jax
import jax.numpy as jnp
from jax import lax
from jax.experimental import pallas as pl
from jax.experimental.pallas import tpu as pltpu
from jax.experimental.pallas import tpu_sc as plsc

_BATCH = 4096
_HIST = 50
_EMBED = 64

_NC = 2
_NS = 16
_NW = _NC * _NS
_BB = _BATCH // _NW
_LANES = 16

_mesh = plsc.VectorSubcoreMesh(core_axis_name="c", subcore_axis_name="s")


@functools.partial(
    pl.kernel,
    mesh=_mesh,
    out_type=jax.ShapeDtypeStruct((_HIST, 8, _NW, 8, 128), jnp.float32),
    compiler_params=pltpu.CompilerParams(
        use_tc_tiling_on_sc=False, needs_layout_passes=False),
    scratch_types=[
        pltpu.VMEM((_BB, _HIST), jnp.int32),
        pltpu.VMEM((_HIST, _BB), jnp.int32),
        pltpu.VMEM((2, _BB, _EMBED), jnp.float32),
        pltpu.VMEM((2, _EMBED, 128), jnp.float32),
        pltpu.SemaphoreType.DMA,
        pltpu.SemaphoreType.DMA,
        pltpu.SemaphoreType.DMA,
        pltpu.SemaphoreType.DMA,
    ],
)
def _gather(idx_hbm, table_hbm, out_hbm, idx_v, idxt_v, rows_v, t_v,
            g0, g1, w0, w1):
    wid = lax.axis_index("s") * _NC + lax.axis_index("c")
    bbase = wid * _BB
    pltpu.sync_copy(idx_hbm.at[pl.ds(bbase, _BB)], idx_v)

    iota = lax.iota(jnp.int32, _LANES)
    bvecs = [iota + bb * _LANES for bb in range(_BB // _LANES)]
    zero16 = jnp.zeros((_LANES,), jnp.int32)

    def idx_t_body(h, c):
        hvec = zero16 + h
        for bb in range(_BB // _LANES):
            raw = plsc.load_gather(idx_v, [bvecs[bb], hvec])
            idxt_v[h, pl.ds(bb * _LANES, _LANES)] = raw + raw
        return c
    lax.fori_loop(0, _HIST, idx_t_body, 0)

    gsem = (g0, g1)
    wsem = (w0, w1)
    kvecs = (zero16, zero16 + 1)

    def start_gather(h, k):
        return pltpu.async_copy(
            table_hbm.at[idxt_v.at[h]], rows_v.at[k], gsem[k])

    def wait_gather(h, k):
        pltpu.make_async_copy(
            table_hbm.at[idxt_v.at[h]], rows_v.at[k], gsem[k]).wait()

    def start_write(h, k):
        for dB in range(8):
            pltpu.async_copy(
                t_v.at[k, pl.ds(dB * 8, 8)], out_hbm.at[h, dB, wid], wsem[k])

    def wait_write(h, k):
        for dB in range(8):
            pltpu.make_async_copy(
                t_v.at[k, pl.ds(dB * 8, 8)], out_hbm.at[h, dB, wid],
                wsem[k]).wait()

    rots = [jnp.bitwise_and(iota + s, 15) for s in range(_LANES)]

    def transpose_rows(h, k):
        kvec = kvecs[k]
        del h
        for b0 in range(0, _BB, _LANES):
            bvec = bvecs[b0 // _LANES]

            @plsc.parallel_loop(0, _EMBED, step=2 * _LANES)
            def _(d0, bvec=bvec):
                for half in range(2):
                    for s in range(_LANES):
                        dvec = rots[s] + (d0 + half * _LANES)
                        v = plsc.load_gather(rows_v, [kvec, bvec, dvec])
                        plsc.store_scatter(t_v, [kvec, dvec, bvec], v)

    start_gather(0, 0)

    def step(i, c):
        h0 = 2 * i
        h1 = h0 + 1
        start_gather(h1, 1)
        wait_gather(h0, 0)

        @pl.when(i > 0)
        def _():
            wait_write(h0, 0)
        transpose_rows(h0, 0)
        start_write(h0, 0)

        @pl.when(i < _HIST // 2 - 1)
        def _():
            start_gather(h0 + 2, 0)
        wait_gather(h1, 1)

        @pl.when(i > 0)
        def _():
            wait_write(h1, 1)
        transpose_rows(h1, 1)
        start_write(h1, 1)
        return c

    lax.fori_loop(0, _HIST // 2, step, 0)
    wait_write(_HIST - 2, 0)
    wait_write(_HIST - 1, 1)


def kernel(inputs, table):
    padded = jnp.pad(table, ((0, 0), (0, 128 - _EMBED)))
    p = _gather(inputs.astype(jnp.int32),
                padded.reshape(2 * 1000000, _EMBED))
    return p.transpose(2, 4, 0, 1, 3).reshape(_BATCH, _HIST, _EMBED)


# speedup vs baseline: 1.0254x; 1.0254x over previous
import functools

import jax
import jax.numpy as jnp
from jax import lax
from jax.experimental import pallas as pl
from jax.experimental.pallas import tpu as pltpu
from jax.experimental.pallas import tpu_sc as plsc

_BATCH = 4096
_HIST = 50
_EMBED = 64

_NC = 2
_NS = 16
_NW = _NC * _NS
_BB = _BATCH // _NW
_LANES = 16

_mesh = plsc.VectorSubcoreMesh(core_axis_name="c", subcore_axis_name="s")


@functools.partial(
    pl.kernel,
    mesh=_mesh,
    out_type=jax.ShapeDtypeStruct((_HIST, 8, _NW, 8, 128), jnp.float32),
    compiler_params=pltpu.CompilerParams(
        use_tc_tiling_on_sc=False, needs_layout_passes=False),
    scratch_types=[
        pltpu.VMEM((_BB, _HIST), jnp.int32),
        pltpu.VMEM((_HIST, _BB), jnp.int32),
        pltpu.VMEM((2, _BB, _EMBED), jnp.float32),
        pltpu.VMEM((2, _EMBED, 128), jnp.float32),
        pltpu.SemaphoreType.DMA,
        pltpu.SemaphoreType.DMA,
        pltpu.SemaphoreType.DMA,
        pltpu.SemaphoreType.DMA,
    ],
)
def _gather(idx_hbm, table_hbm, out_hbm, idx_v, idxt_v, rows_v, t_v,
            g0, g1, w0, w1):
    wid = lax.axis_index("s") * _NC + lax.axis_index("c")
    bbase = wid * _BB
    pltpu.sync_copy(idx_hbm.at[pl.ds(bbase, _BB)], idx_v)

    iota = lax.iota(jnp.int32, _LANES)
    bvecs = [iota + bb * _LANES for bb in range(_BB // _LANES)]
    zero16 = jnp.zeros((_LANES,), jnp.int32)

    def idx_t_body(h, c):
        hvec = zero16 + h
        for bb in range(_BB // _LANES):
            raw = plsc.load_gather(idx_v, [bvecs[bb], hvec])
            idxt_v[h, pl.ds(bb * _LANES, _LANES)] = raw + raw
        return c
    lax.fori_loop(0, _HIST, idx_t_body, 0)

    gsem = (g0, g1)
    wsem = (w0, w1)
    kvecs = (zero16, zero16 + 1)

    def start_gather(h, k):
        return pltpu.async_copy(
            table_hbm.at[idxt_v.at[h]], rows_v.at[k], gsem[k])

    def wait_gather(h, k):
        pltpu.make_async_copy(
            table_hbm.at[idxt_v.at[h]], rows_v.at[k], gsem[k]).wait()

    def start_write(h, k):
        for dB in range(8):
            pltpu.async_copy(
                t_v.at[k, pl.ds(dB * 8, 8)], out_hbm.at[h, dB, wid], wsem[k])

    def wait_write(h, k):
        for dB in range(8):
            pltpu.make_async_copy(
                t_v.at[k, pl.ds(dB * 8, 8)], out_hbm.at[h, dB, wid],
                wsem[k]).wait()

    rots = [jnp.bitwise_and(iota + s, 15) for s in range(_LANES)]

    def transpose_rows(h, k):
        kvec = kvecs[k]
        del h

        @plsc.parallel_loop(0, _BB, step=_LANES, unroll=2)
        def _(b0):
            bvec = iota + b0
            for d0 in range(0, _EMBED, _LANES):
                for s in range(_LANES):
                    dvec = rots[s] + d0
                    v = plsc.load_gather(rows_v, [kvec, bvec, dvec])
                    plsc.store_scatter(t_v, [kvec, dvec, bvec], v)

    start_gather(0, 0)

    def step(i, c):
        h0 = 2 * i
        h1 = h0 + 1
        start_gather(h1, 1)
        wait_gather(h0, 0)

        @pl.when(i > 0)
        def _():
            wait_write(h0, 0)
        transpose_rows(h0, 0)
        start_write(h0, 0)

        @pl.when(i < _HIST // 2 - 1)
        def _():
            start_gather(h0 + 2, 0)
        wait_gather(h1, 1)

        @pl.when(i > 0)
        def _():
            wait_write(h1, 1)
        transpose_rows(h1, 1)
        start_write(h1, 1)
        return c

    lax.fori_loop(0, _HIST // 2, step, 0)
    wait_write(_HIST - 2, 0)
    wait_write(_HIST - 1, 1)


def kernel(inputs, table):
    padded = jnp.pad(table, ((0, 0), (0, 128 - _EMBED)))
    p = _gather(inputs.astype(jnp.int32),
                padded.reshape(2 * 1000000, _EMBED))
    return p.transpose(2, 4, 0, 1, 3).reshape(_BATCH, _HIST, _EMBED)


# speedup vs baseline: 1.1047x; 1.0773x over previous
import functools

import jax
import jax.numpy as jnp
from jax import lax
from jax.experimental import pallas as pl
from jax.experimental.pallas import tpu as pltpu
from jax.experimental.pallas import tpu_sc as plsc

_BATCH = 4096
_HIST = 50
_EMBED = 64

_NC = 2
_NS = 16
_NW = _NC * _NS
_BB = _BATCH // _NW
_LANES = 16

_mesh = plsc.VectorSubcoreMesh(core_axis_name="c", subcore_axis_name="s")


@functools.partial(
    pl.kernel,
    mesh=_mesh,
    out_type=jax.ShapeDtypeStruct((_HIST, 8, _NW, 8, 128), jnp.float32),
    compiler_params=pltpu.CompilerParams(
        use_tc_tiling_on_sc=False, needs_layout_passes=False),
    scratch_types=[
        pltpu.VMEM((_BB, _HIST), jnp.int32),
        pltpu.VMEM((_HIST, _BB), jnp.int32),
        pltpu.VMEM((2, _BB, _EMBED), jnp.float32),
        pltpu.VMEM((2, _EMBED, 128), jnp.float32),
        pltpu.SemaphoreType.DMA,
        pltpu.SemaphoreType.DMA,
        pltpu.SemaphoreType.DMA,
        pltpu.SemaphoreType.DMA,
    ],
)
def _gather(idx_hbm, table_hbm, out_hbm, idx_v, idxt_v, rows_v, t_v,
            g0, g1, w0, w1):
    wid = lax.axis_index("s") * _NC + lax.axis_index("c")
    bbase = wid * _BB
    pltpu.sync_copy(idx_hbm.at[pl.ds(bbase, _BB)], idx_v)

    iota = lax.iota(jnp.int32, _LANES)
    bvecs = [iota + bb * _LANES for bb in range(_BB // _LANES)]
    zero16 = jnp.zeros((_LANES,), jnp.int32)

    def idx_t_body(h, c):
        hvec = zero16 + h
        for bb in range(_BB // _LANES):
            raw = plsc.load_gather(idx_v, [bvecs[bb], hvec])
            idxt_v[h, pl.ds(bb * _LANES, _LANES)] = raw + raw
        return c
    lax.fori_loop(0, _HIST, idx_t_body, 0)

    gsem = (g0, g1)
    wsem = (w0, w1)
    kvecs = (zero16, zero16 + 1)

    def start_gather(h, k):
        return pltpu.async_copy(
            table_hbm.at[idxt_v.at[h]], rows_v.at[k], gsem[k])

    def wait_gather(h, k):
        pltpu.make_async_copy(
            table_hbm.at[idxt_v.at[h]], rows_v.at[k], gsem[k]).wait()

    def start_write(h, k):
        for dB in range(8):
            pltpu.async_copy(
                t_v.at[k, pl.ds(dB * 8, 8)], out_hbm.at[h, dB, wid], wsem[k])

    def wait_write(h, k):
        for dB in range(8):
            pltpu.make_async_copy(
                t_v.at[k, pl.ds(dB * 8, 8)], out_hbm.at[h, dB, wid],
                wsem[k]).wait()

    rots = [jnp.bitwise_and(iota + s, 15) for s in range(_LANES)]

    def transpose_rows(h, k):
        kvec = kvecs[k]
        del h

        @plsc.parallel_loop(0, _BB, step=_LANES, unroll=4)
        def _(b0):
            bvec = iota + b0
            for d0 in range(0, _EMBED, _LANES):
                for s in range(_LANES):
                    dvec = rots[s] + d0
                    v = plsc.load_gather(rows_v, [kvec, bvec, dvec])
                    plsc.store_scatter(t_v, [kvec, dvec, bvec], v)

    start_gather(0, 0)

    def step(i, c):
        h0 = 2 * i
        h1 = h0 + 1
        start_gather(h1, 1)
        wait_gather(h0, 0)

        @pl.when(i > 0)
        def _():
            wait_write(h0, 0)
        transpose_rows(h0, 0)
        start_write(h0, 0)

        @pl.when(i < _HIST // 2 - 1)
        def _():
            start_gather(h0 + 2, 0)
        wait_gather(h1, 1)

        @pl.when(i > 0)
        def _():
            wait_write(h1, 1)
        transpose_rows(h1, 1)
        start_write(h1, 1)
        return c

    lax.fori_loop(0, _HIST // 2, step, 0)
    wait_write(_HIST - 2, 0)
    wait_write(_HIST - 1, 1)


def kernel(inputs, table):
    padded = jnp.pad(table, ((0, 0), (0, 128 - _EMBED)))
    p = _gather(inputs.astype(jnp.int32),
                padded.reshape(2 * 1000000, _EMBED))
    return p.transpose(2, 4, 0, 1, 3).reshape(_BATCH, _HIST, _EMBED)


# speedup vs baseline: 1.1079x; 1.0028x over previous
import functools

import jax
import jax.numpy as jnp
from jax import lax
from jax.experimental import pallas as pl
from jax.experimental.pallas import tpu as pltpu
from jax.experimental.pallas import tpu_sc as plsc

_BATCH = 4096
_HIST = 50
_EMBED = 64

_NC = 2
_NS = 16
_NW = _NC * _NS
_BB = _BATCH // _NW
_LANES = 16

_mesh = plsc.VectorSubcoreMesh(core_axis_name="c", subcore_axis_name="s")


@functools.partial(
    pl.kernel,
    mesh=_mesh,
    out_type=jax.ShapeDtypeStruct((_HIST, 8, _NW, 8, 128), jnp.float32),
    compiler_params=pltpu.CompilerParams(
        use_tc_tiling_on_sc=False, needs_layout_passes=False),
    scratch_types=[
        pltpu.VMEM((_BB, _HIST), jnp.int32),
        pltpu.VMEM((_HIST, _BB), jnp.int32),
        pltpu.VMEM((2, _BB, _EMBED), jnp.float32),
        pltpu.VMEM((2, _EMBED, 128), jnp.float32),
        pltpu.SemaphoreType.DMA,
        pltpu.SemaphoreType.DMA,
        pltpu.SemaphoreType.DMA,
        pltpu.SemaphoreType.DMA,
    ],
)
def _gather(idx_hbm, table_hbm, out_hbm, idx_v, idxt_v, rows_v, t_v,
            g0, g1, w0, w1):
    wid = lax.axis_index("s") * _NC + lax.axis_index("c")
    bbase = wid * _BB
    pltpu.sync_copy(idx_hbm.at[pl.ds(bbase, _BB)], idx_v)

    iota = lax.iota(jnp.int32, _LANES)
    bvecs = [iota + bb * _LANES for bb in range(_BB // _LANES)]
    zero16 = jnp.zeros((_LANES,), jnp.int32)

    @plsc.parallel_loop(0, _HIST, unroll=4)
    def _(h):
        hvec = zero16 + h
        for bb in range(_BB // _LANES):
            raw = plsc.load_gather(idx_v, [bvecs[bb], hvec])
            idxt_v[h, pl.ds(bb * _LANES, _LANES)] = raw + raw

    gsem = (g0, g1)
    wsem = (w0, w1)
    kvecs = (zero16, zero16 + 1)

    def start_gather(h, k):
        return pltpu.async_copy(
            table_hbm.at[idxt_v.at[h]], rows_v.at[k], gsem[k])

    def wait_gather(h, k):
        pltpu.make_async_copy(
            table_hbm.at[idxt_v.at[h]], rows_v.at[k], gsem[k]).wait()

    def start_write(h, k):
        for dB in range(8):
            pltpu.async_copy(
                t_v.at[k, pl.ds(dB * 8, 8)], out_hbm.at[h, dB, wid], wsem[k])

    def wait_write(h, k):
        for dB in range(8):
            pltpu.make_async_copy(
                t_v.at[k, pl.ds(dB * 8, 8)], out_hbm.at[h, dB, wid],
                wsem[k]).wait()

    rots = [jnp.bitwise_and(iota + s, 15) for s in range(_LANES)]

    def transpose_rows(h, k):
        kvec = kvecs[k]
        del h

        @plsc.parallel_loop(0, _BB, step=_LANES, unroll=4)
        def _(b0):
            bvec = iota + b0
            for d0 in range(0, _EMBED, _LANES):
                for s in range(_LANES):
                    dvec = rots[s] + d0
                    v = plsc.load_gather(rows_v, [kvec, bvec, dvec])
                    plsc.store_scatter(t_v, [kvec, dvec, bvec], v)

    start_gather(0, 0)

    def step(i, c):
        h0 = 2 * i
        h1 = h0 + 1
        start_gather(h1, 1)
        wait_gather(h0, 0)

        @pl.when(i > 0)
        def _():
            wait_write(h0, 0)
        transpose_rows(h0, 0)
        start_write(h0, 0)

        @pl.when(i < _HIST // 2 - 1)
        def _():
            start_gather(h0 + 2, 0)
        wait_gather(h1, 1)

        @pl.when(i > 0)
        def _():
            wait_write(h1, 1)
        transpose_rows(h1, 1)
        start_write(h1, 1)
        return c

    lax.fori_loop(0, _HIST // 2, step, 0)
    wait_write(_HIST - 2, 0)
    wait_write(_HIST - 1, 1)


def kernel(inputs, table):
    padded = jnp.pad(table, ((0, 0), (0, 128 - _EMBED)))
    p = _gather(inputs.astype(jnp.int32),
                padded.reshape(2 * 1000000, _EMBED))
    return p.transpose(2, 4, 0, 1, 3).reshape(_BATCH, _HIST, _EMBED)
